# baseline (speedup 1.0000x reference)
import functools
import numpy as np
import jax
import jax.numpy as jnp
from jax import lax
from jax.experimental import pallas as pl
from jax.experimental.pallas import tpu as pltpu
from jax.experimental.pallas import tpu_sc as plsc

_SUPPORT = 300
_EPS = 0.001
_N = 2 * _SUPPORT + 1
_NP = 608
_E = 32768
_NW = 32
_EW = _E // _NW
_CE = 64
_NCH = _EW // _CE
_NG = _CE // 16
_NBUF = 2


def _np_decode(y):
    y = np.asarray(y, np.float32)
    eps = np.float32(_EPS)
    one = np.float32(1.0)
    return np.sign(y) * (
        np.square(
            (np.sqrt(one + np.float32(4.0) * eps * (np.abs(y) + one + eps)) - one)
            / (np.float32(2.0) * eps)
        )
        - one
    )


_A_LO = float(_np_decode(-_SUPPORT))
_A_HI = float(_np_decode(_SUPPORT))


def _dec_jnp(y):
    return jnp.sign(y) * (
        jnp.square(
            (jnp.sqrt(1.0 + 4.0 * _EPS * (jnp.abs(y) + 1.0 + _EPS)) - 1.0)
            / (2.0 * _EPS)
        )
        - 1.0
    )


def _sqrt16(t):
    yi = lax.bitcast_convert_type(t, jnp.int32)
    g = lax.bitcast_convert_type(
        lax.shift_right_arithmetic(yi, 1) + jnp.int32(0x1FBD1DF5), jnp.float32
    )
    g = 0.5 * (g + t / g)
    g = 0.5 * (g + t / g)
    g = 0.5 * (g + t / g)
    return g


def _bin16(xv, atbuf):
    xc = jnp.clip(xv, jnp.float32(_A_LO), jnp.float32(_A_HI))
    ax = jnp.abs(xc)
    f = jnp.sign(xc) * (_sqrt16(ax + 1.0) - 1.0 + jnp.float32(_EPS) * ax)
    k = (f + jnp.float32(_SUPPORT)).astype(jnp.int32)
    k = jnp.clip(k, 0, _N - 2)
    a = plsc.load_gather(atbuf, [k])
    b = plsc.load_gather(atbuf, [k + 1])
    k = jnp.where(b < xc, k + 1, jnp.where(xc <= a, k - 1, k))
    k = jnp.clip(k, 0, _N - 2)
    lb = plsc.load_gather(atbuf, [k])
    ub = plsc.load_gather(atbuf, [k + 1])
    ld = (ub - xc) / (ub - lb)
    return k, ld, 1.0 - ld


def _sc_body(x_hbm, atoms_hbm, zeros_hbm, out_hbm, xbuf, atbuf, buf0, buf1,
             kidx, sem0, sem1):
    cid = lax.axis_index("c")
    sid = lax.axis_index("s")
    w = sid * 2 + cid
    e0 = w * _EW

    pltpu.sync_copy(x_hbm.at[pl.ds(e0, _EW)], xbuf)
    pltpu.sync_copy(atoms_hbm, atbuf)
    pltpu.sync_copy(zeros_hbm, buf0)
    pltpu.sync_copy(zeros_hbm, buf1)

    bufs = (buf0, buf1)
    sems = (sem0, sem1)
    lanes = lax.iota(jnp.int32, 16)
    z16 = jnp.zeros((16,), jnp.float32)
    for b in range(_NBUF):
        for g in range(_NG):
            kidx[pl.ds((b * _NG + g) * 16, 16)] = jnp.zeros((16,), jnp.int32)

    def chunk_copy(buf, chunk_e0, sem):
        return pltpu.make_async_copy(
            buf, out_hbm.at[pl.ds(chunk_e0, _CE)], sem
        )

    def step(ci, carry):
        for b in range(_NBUF):
            chunk = ci * _NBUF + b
            buf = bufs[b]
            sem = sems[b]
            chunk_e0 = e0 + chunk * _CE

            @pl.when(chunk >= _NBUF)
            def _retire():
                chunk_copy(buf, chunk_e0 - _NBUF * _CE, sem).wait()

            for g in range(_NG):
                rows = lanes + g * 16
                kv = kidx[pl.ds((b * _NG + g) * 16, 16)]
                plsc.store_scatter(buf, [rows, kv], z16)
                plsc.store_scatter(buf, [rows, kv + 1], z16)

            for g in range(_NG):
                xv = xbuf[pl.ds(chunk * _CE + g * 16, 16)]
                k, ld, ud = _bin16(xv, atbuf)
                rows = lanes + g * 16
                plsc.store_scatter(buf, [rows, k], ld)
                plsc.store_scatter(buf, [rows, k + 1], ud)
                kidx[pl.ds((b * _NG + g) * 16, 16)] = k

            chunk_copy(buf, chunk_e0, sem).start()
        return carry

    lax.fori_loop(0, _NCH // _NBUF, step, 0)

    for b in range(_NBUF):
        chunk = _NCH - _NBUF + b
        chunk_copy(bufs[b], e0 + chunk * _CE, sems[b]).wait()


@jax.jit
def _sc_twohot(x_flat, atoms, zeros_buf):
    mesh = plsc.VectorSubcoreMesh(
        core_axis_name="c", subcore_axis_name="s", num_cores=2, num_subcores=16
    )
    f = pl.kernel(
        _sc_body,
        mesh=mesh,
        compiler_params=pltpu.CompilerParams(needs_layout_passes=False),
        out_type=jax.ShapeDtypeStruct((_E, _N), jnp.float32),
        scratch_types=[
            pltpu.VMEM((_EW,), jnp.float32),
            pltpu.VMEM((_N,), jnp.float32),
            pltpu.VMEM((_CE, _N), jnp.float32),
            pltpu.VMEM((_CE, _N), jnp.float32),
            pltpu.VMEM((_NBUF * _CE,), jnp.int32),
            pltpu.SemaphoreType.DMA,
            pltpu.SemaphoreType.DMA,
        ],
    )
    return f(x_flat, atoms, zeros_buf)


def kernel(x):
    atoms = _dec_jnp(jnp.arange(-_SUPPORT, _SUPPORT + 1, dtype=jnp.float32))
    zeros_buf = jnp.zeros((_CE, _N), jnp.float32)
    xf = x.reshape(2 * _E)
    a = _sc_twohot(xf[:_E], atoms, zeros_buf)
    b = _sc_twohot(xf[_E:], atoms, zeros_buf)
    rows = x.shape[0] // 2
    return jnp.concatenate(
        [a.reshape(rows, x.shape[1], _N), b.reshape(rows, x.shape[1], _N)], 0
    )


# speedup vs baseline: 1.5727x; 1.5727x over previous
import functools
import numpy as np
import jax
import jax.numpy as jnp
from jax import lax
from jax.experimental import pallas as pl
from jax.experimental.pallas import tpu as pltpu
from jax.experimental.pallas import tpu_sc as plsc

_SUPPORT = 300
_EPS = 0.001
_N = 2 * _SUPPORT + 1
_NP = 608
_E = 65536
_NW = 32
_EW = _E // _NW
_CE = 64
_NCH = _EW // _CE
_NG = _CE // 16
_NBUF = 2


def _np_decode(y):
    y = np.asarray(y, np.float32)
    eps = np.float32(_EPS)
    one = np.float32(1.0)
    return np.sign(y) * (
        np.square(
            (np.sqrt(one + np.float32(4.0) * eps * (np.abs(y) + one + eps)) - one)
            / (np.float32(2.0) * eps)
        )
        - one
    )


_A_LO = float(_np_decode(-_SUPPORT))
_A_HI = float(_np_decode(_SUPPORT))


def _dec_jnp(y):
    return jnp.sign(y) * (
        jnp.square(
            (jnp.sqrt(1.0 + 4.0 * _EPS * (jnp.abs(y) + 1.0 + _EPS)) - 1.0)
            / (2.0 * _EPS)
        )
        - 1.0
    )


def _sqrt16(t):
    yi = lax.bitcast_convert_type(t, jnp.int32)
    g = lax.bitcast_convert_type(
        lax.shift_right_arithmetic(yi, 1) + jnp.int32(0x1FBD1DF5), jnp.float32
    )
    g = 0.5 * (g + t / g)
    g = 0.5 * (g + t / g)
    g = 0.5 * (g + t / g)
    return g


def _bin16(xv, atbuf):
    xc = jnp.clip(xv, jnp.float32(_A_LO), jnp.float32(_A_HI))
    ax = jnp.abs(xc)
    f = jnp.sign(xc) * (_sqrt16(ax + 1.0) - 1.0 + jnp.float32(_EPS) * ax)
    k = (f + jnp.float32(_SUPPORT)).astype(jnp.int32)
    k = jnp.clip(k, 0, _N - 2)
    a = plsc.load_gather(atbuf, [k])
    b = plsc.load_gather(atbuf, [k + 1])
    k = jnp.where(b < xc, k + 1, jnp.where(xc <= a, k - 1, k))
    k = jnp.clip(k, 0, _N - 2)
    lb = plsc.load_gather(atbuf, [k])
    ub = plsc.load_gather(atbuf, [k + 1])
    ld = (ub - xc) / (ub - lb)
    return k, ld, 1.0 - ld


def _sc_body(x_hbm, atoms_hbm, zeros_hbm, out_hbm, xbuf, atbuf, buf0, buf1,
             kidx, sem0, sem1):
    cid = lax.axis_index("c")
    sid = lax.axis_index("s")
    w = sid * 2 + cid
    e0 = w * _EW

    pltpu.sync_copy(x_hbm.at[pl.ds(e0, _EW)], xbuf)
    pltpu.sync_copy(atoms_hbm, atbuf)
    pltpu.sync_copy(zeros_hbm, buf0)
    pltpu.sync_copy(zeros_hbm, buf1)

    bufs = (buf0, buf1)
    sems = (sem0, sem1)
    lanes = lax.iota(jnp.int32, 16)
    z16 = jnp.zeros((16,), jnp.float32)
    for b in range(_NBUF):
        for g in range(_NG):
            kidx[pl.ds((b * _NG + g) * 16, 16)] = jnp.zeros((16,), jnp.int32)

    def chunk_copy(buf, chunk_e0, sem):
        return pltpu.make_async_copy(
            buf, out_hbm.at[pl.ds(chunk_e0, _CE)], sem
        )

    def step(ci, carry):
        for b in range(_NBUF):
            chunk = ci * _NBUF + b
            buf = bufs[b]
            sem = sems[b]
            chunk_e0 = e0 + chunk * _CE

            @pl.when(chunk >= _NBUF)
            def _retire():
                chunk_copy(buf, chunk_e0 - _NBUF * _CE, sem).wait()

            for g in range(_NG):
                rows = lanes + g * 16
                kv = kidx[pl.ds((b * _NG + g) * 16, 16)]
                plsc.store_scatter(buf, [rows, kv], z16)
                plsc.store_scatter(buf, [rows, kv + 1], z16)

            for g in range(_NG):
                xv = xbuf[pl.ds(chunk * _CE + g * 16, 16)]
                k, ld, ud = _bin16(xv, atbuf)
                rows = lanes + g * 16
                plsc.store_scatter(buf, [rows, k], ld)
                plsc.store_scatter(buf, [rows, k + 1], ud)
                kidx[pl.ds((b * _NG + g) * 16, 16)] = k

            chunk_copy(buf, chunk_e0, sem).start()
        return carry

    lax.fori_loop(0, _NCH // _NBUF, step, 0)

    for b in range(_NBUF):
        chunk = _NCH - _NBUF + b
        chunk_copy(bufs[b], e0 + chunk * _CE, sems[b]).wait()


@jax.jit
def _sc_twohot(x_flat, atoms, zeros_buf):
    mesh = plsc.VectorSubcoreMesh(
        core_axis_name="c", subcore_axis_name="s", num_cores=2, num_subcores=16
    )
    f = pl.kernel(
        _sc_body,
        mesh=mesh,
        compiler_params=pltpu.CompilerParams(needs_layout_passes=False),
        out_type=jax.ShapeDtypeStruct((_E, _N), jnp.float32),
        scratch_types=[
            pltpu.VMEM((_EW,), jnp.float32),
            pltpu.VMEM((_N,), jnp.float32),
            pltpu.VMEM((_CE, _N), jnp.float32),
            pltpu.VMEM((_CE, _N), jnp.float32),
            pltpu.VMEM((_NBUF * _CE,), jnp.int32),
            pltpu.SemaphoreType.DMA,
            pltpu.SemaphoreType.DMA,
        ],
    )
    return f(x_flat, atoms, zeros_buf)


def kernel(x):
    atoms = _dec_jnp(jnp.arange(-_SUPPORT, _SUPPORT + 1, dtype=jnp.float32))
    zeros_buf = jnp.zeros((_CE, _N), jnp.float32)
    out = _sc_twohot(x.reshape(_E), atoms, zeros_buf)
    return out.reshape(x.shape[0], x.shape[1], _N)
